# baseline (speedup 1.0000x reference)
import functools

import jax
import jax.numpy as jnp
from jax.experimental import pallas as pl
from jax.experimental.pallas import tpu as pltpu

N = 10000
R = 16
H = 256
C = 256
E = 160000

BLK = 400


def _einsum_block(agg_ref, w2_ref, h_ref, root2_ref, bias2_ref, out_ref):
    acc = jnp.dot(h_ref[...], root2_ref[...], preferred_element_type=jnp.float32,
                  precision=jax.lax.Precision.HIGHEST)

    def body(r, acc):
        return acc + jnp.dot(agg_ref[r], w2_ref[r],
                             preferred_element_type=jnp.float32,
                             precision=jax.lax.Precision.HIGHEST)

    acc = jax.lax.fori_loop(0, R, body, acc)
    acc = acc + bias2_ref[...]
    m = jnp.max(acc, axis=1, keepdims=True)
    s = jnp.log(jnp.sum(jnp.exp(acc - m), axis=1, keepdims=True))
    out_ref[...] = acc - m - s


def _fused_tail(agg, w2, h, root2, bias2):
    grid = (N // BLK,)
    return pl.pallas_call(
        _einsum_block,
        grid=grid,
        in_specs=[
            pl.BlockSpec((R, BLK, H), lambda i: (0, i, 0)),
            pl.BlockSpec((R, H, C), lambda i: (0, 0, 0)),
            pl.BlockSpec((BLK, H), lambda i: (i, 0)),
            pl.BlockSpec((H, C), lambda i: (0, 0)),
            pl.BlockSpec((1, C), lambda i: (0, 0)),
        ],
        out_specs=pl.BlockSpec((BLK, C), lambda i: (i, 0)),
        out_shape=jax.ShapeDtypeStruct((N, C), jnp.float32),
    )(agg, w2, h, root2, bias2.reshape(1, C))


def kernel(edge_index, edge_attr, weight1, root1, bias1, weight2, root2, bias2):
    src = edge_index[0]
    dst = edge_index[1]
    edge_type = edge_attr
    comb = edge_type * N + dst
    cnt = jnp.zeros((R * N,), jnp.int32).at[comb].add(1)
    inv = 1.0 / jnp.maximum(cnt, 1).astype(jnp.float32)
    en = inv[comb]
    msg1 = weight1.reshape(R * N, H)[edge_type * N + src] * en[:, None]
    h = jnp.zeros((N, H), jnp.float32).at[dst].add(msg1)
    h = jax.nn.relu(h + root1 + bias1)
    xs = h[src] * en[:, None]
    agg = jnp.zeros((R * N, H), jnp.float32).at[comb].add(xs).reshape(R, N, H)
    return _fused_tail(agg, weight2, h, root2, bias2)


# speedup vs baseline: 1.8532x; 1.8532x over previous
import functools

import jax
import jax.numpy as jnp
from jax import lax
from jax.experimental import pallas as pl
from jax.experimental.pallas import tpu as pltpu
from jax.experimental.pallas import tpu_sc as plsc

N = 10000
R = 16
H = 256
C = 256
E = 160000

NC = 2
NS = 16
L = 16
HALF = N // NC
ACC_ROWS = 5120
TRASH = HALF
STRIPE = ACC_ROWS // NS
EPS = E // NS
BB = 400
NBLK = EPS // BB
W = 256
HW = W // 2

_mesh = plsc.VectorSubcoreMesh(core_axis_name="c", subcore_axis_name="s")


def _fill(ref, n, value):
    v = jnp.full((L,), value, ref.dtype)

    def body(i, _):
        ref[pl.ds(i * L, L)] = v
        return 0

    lax.fori_loop(0, n // L, body, 0)


def _en_body(rel_hbm, dst_hbm, en_hbm, rel_v, dst_v, comb_v, ones_v, cnt_v,
             hist_sh, sem):
    c = lax.axis_index("c")
    s = lax.axis_index("s")
    base = s * EPS
    _fill(cnt_v, EPS, 0.0)
    _fill(ones_v, EPS, 1.0)
    pltpu.sync_copy(cnt_v, hist_sh.at[pl.ds(s * (R * N // NS), R * N // NS)])
    pltpu.sync_copy(rel_hbm.at[pl.ds(base, EPS)], rel_v)
    pltpu.sync_copy(dst_hbm.at[pl.ds(base, EPS)], dst_v)

    def comb(i, _):
        sl = pl.ds(i * L, L)
        comb_v[sl] = rel_v[sl] * N + dst_v[sl]
        return 0

    lax.fori_loop(0, EPS // L, comb, 0)
    plsc.subcore_barrier()
    pltpu.sync_copy(ones_v, hist_sh.at[comb_v], add=True)
    plsc.subcore_barrier()
    pltpu.async_copy(hist_sh.at[comb_v], cnt_v, sem).wait()

    def inv(i, _):
        sl = pl.ds(i * L, L)
        cnt_v[sl] = 1.0 / jnp.maximum(cnt_v[sl], 1.0)
        return 0

    lax.fori_loop(0, EPS // L, inv, 0)

    @pl.when(c == 0)
    def _():
        pltpu.sync_copy(cnt_v, en_hbm.at[pl.ds(base, EPS)])


@functools.partial(
    pl.kernel,
    out_type=jax.ShapeDtypeStruct((E,), jnp.float32),
    mesh=_mesh,
    scratch_types=[
        pltpu.VMEM((EPS,), jnp.int32),
        pltpu.VMEM((EPS,), jnp.int32),
        pltpu.VMEM((EPS,), jnp.int32),
        pltpu.VMEM((EPS,), jnp.float32),
        pltpu.VMEM((EPS,), jnp.float32),
        pltpu.VMEM_SHARED((R * N,), jnp.float32),
        pltpu.SemaphoreType.DMA,
    ],
)
def _en_sc(rel_hbm, dst_hbm, en_hbm, *rest):
    _en_body(rel_hbm, dst_hbm, en_hbm, *rest)


def _gs_body(table_hbm, rel_hbm, src_hbm, dst_hbm, en_hbm, out_hbm,
             ga_v, src_v, dst_v, en_v, rows_v, acc_sh, sem):
    c = lax.axis_index("c")
    s = lax.axis_index("s")
    row0 = c * HALF
    rows2 = rows_v

    for jh in range(2):
        def zrow(i, _):
            for j in range(HW // L):
                rows2[i, pl.ds(j * L, L)] = jnp.zeros((L,), jnp.float32)
            return 0

        lax.fori_loop(0, STRIPE, zrow, 0)
        pltpu.sync_copy(rows2.at[pl.ds(0, STRIPE)],
                        acc_sh.at[pl.ds(s * STRIPE, STRIPE)])
        plsc.subcore_barrier()

        def block(b, _):
            base = s * EPS + b * BB
            pltpu.sync_copy(rel_hbm.at[pl.ds(base, BB)], ga_v)
            pltpu.sync_copy(src_hbm.at[pl.ds(base, BB)], src_v)
            pltpu.sync_copy(dst_hbm.at[pl.ds(base, BB)], dst_v)
            pltpu.sync_copy(en_hbm.at[pl.ds(base, BB)], en_v)

            def prep(i, _):
                sl = pl.ds(i * L, L)
                idx2 = (ga_v[sl] * N + src_v[sl]) * 2
                ga_v[sl] = idx2 + jh
                d = dst_v[sl] - row0
                own = (d >= 0) & (d < HALF)
                dst_v[sl] = jnp.where(own, d, TRASH)
                return 0

            lax.fori_loop(0, BB // L, prep, 0)
            pltpu.async_copy(table_hbm.at[ga_v], rows2, sem).wait()

            def scale(g, _):
                ev16 = en_v[pl.ds(g * L, L)]
                for r in range(L):
                    ev = lax.broadcast_in_dim(ev16[r], (L,), ())
                    i = g * L + r
                    for j in range(HW // L):
                        sl = pl.ds(j * L, L)
                        rows2[i, sl] = rows2[i, sl] * ev
                return 0

            lax.fori_loop(0, BB // L, scale, 0)
            pltpu.sync_copy(rows2, acc_sh.at[dst_v], add=True)
            return 0

        lax.fori_loop(0, NBLK, block, 0)
        plsc.subcore_barrier()

        @pl.when(s < NS - 1)
        def _():
            pltpu.sync_copy(
                acc_sh.at[pl.ds(s * STRIPE, STRIPE)],
                out_hbm.at[pl.ds(row0 + s * STRIPE, STRIPE),
                           pl.ds(jh * HW, HW)])

        @pl.when(s == NS - 1)
        def _():
            last = HALF - (NS - 1) * STRIPE
            pltpu.sync_copy(
                acc_sh.at[pl.ds((NS - 1) * STRIPE, last)],
                out_hbm.at[pl.ds(row0 + (NS - 1) * STRIPE, last),
                           pl.ds(jh * HW, HW)])

        plsc.subcore_barrier()


@functools.partial(
    pl.kernel,
    out_type=jax.ShapeDtypeStruct((N, W), jnp.float32),
    mesh=_mesh,
    scratch_types=[
        pltpu.VMEM((BB,), jnp.int32),
        pltpu.VMEM((BB,), jnp.int32),
        pltpu.VMEM((BB,), jnp.int32),
        pltpu.VMEM((BB,), jnp.float32),
        pltpu.VMEM((BB, HW), jnp.float32),
        pltpu.VMEM_SHARED((ACC_ROWS, HW), jnp.float32),
        pltpu.SemaphoreType.DMA,
    ],
)
def _gs_sc(table_hbm, rel_hbm, src_hbm, dst_hbm, en_hbm, out_hbm, *rest):
    _gs_body(table_hbm, rel_hbm, src_hbm, dst_hbm, en_hbm, out_hbm, *rest)


BLK = 400


def _h_block(acc1_ref, root1_ref, bias1_ref):
    h = acc1_ref[...] + root1_ref[...] + bias1_ref[...]
    return jnp.maximum(h, 0.0).astype(jnp.bfloat16)


def _y_block(acc1_ref, root1_ref, bias1_ref, w2_ref, y_ref):
    h = _h_block(acc1_ref, root1_ref, bias1_ref)

    def body(r, _):
        y_ref[r] = jnp.dot(h, w2_ref[r], preferred_element_type=jnp.float32)
        return 0

    lax.fori_loop(0, R, body, 0, unroll=True)


def _y_tc(acc1, root1, bias1, w2_bf16):
    return pl.pallas_call(
        _y_block,
        grid=(N // BLK,),
        in_specs=[
            pl.BlockSpec((BLK, H), lambda i: (i, 0)),
            pl.BlockSpec((BLK, H), lambda i: (i, 0)),
            pl.BlockSpec((1, H), lambda i: (0, 0)),
            pl.BlockSpec((R, H, C), lambda i: (0, 0, 0)),
        ],
        out_specs=pl.BlockSpec((R, BLK, C), lambda i: (0, i, 0)),
        out_shape=jax.ShapeDtypeStruct((R, N, C), jnp.float32),
    )(acc1, root1, bias1, w2_bf16)


def _tail_block(acc2_ref, acc1_ref, root1_ref, bias1_ref, root2_ref,
                bias2_ref, out_ref):
    h = _h_block(acc1_ref, root1_ref, bias1_ref)
    o = acc2_ref[...] + jnp.dot(h, root2_ref[...],
                                preferred_element_type=jnp.float32)
    o = o + bias2_ref[...]
    m = jnp.max(o, axis=1, keepdims=True)
    lse = jnp.log(jnp.sum(jnp.exp(o - m), axis=1, keepdims=True))
    out_ref[...] = o - m - lse


def _tail_tc(acc2, acc1, root1, bias1, root2_bf16, bias2):
    return pl.pallas_call(
        _tail_block,
        grid=(N // BLK,),
        in_specs=[
            pl.BlockSpec((BLK, C), lambda i: (i, 0)),
            pl.BlockSpec((BLK, H), lambda i: (i, 0)),
            pl.BlockSpec((BLK, H), lambda i: (i, 0)),
            pl.BlockSpec((1, H), lambda i: (0, 0)),
            pl.BlockSpec((H, C), lambda i: (0, 0)),
            pl.BlockSpec((1, C), lambda i: (0, 0)),
        ],
        out_specs=pl.BlockSpec((BLK, C), lambda i: (i, 0)),
        out_shape=jax.ShapeDtypeStruct((N, C), jnp.float32),
    )(acc2, acc1, root1, bias1, root2_bf16, bias2)


def kernel(edge_index, edge_attr, weight1, root1, bias1, weight2, root2, bias2):
    src = edge_index[0].astype(jnp.int32)
    dst = edge_index[1].astype(jnp.int32)
    rel = edge_attr.astype(jnp.int32)
    en = _en_sc(rel, dst)
    acc1 = _gs_sc(weight1.reshape(R * N * 2, HW), rel, src, dst, en)
    y = _y_tc(acc1, root1, bias1.reshape(1, H), weight2.astype(jnp.bfloat16))
    acc2 = _gs_sc(y.reshape(R * N * 2, HW), rel, src, dst, en)
    return _tail_tc(acc2, acc1, root1, bias1.reshape(1, H),
                    root2.astype(jnp.bfloat16), bias2.reshape(1, C))


# speedup vs baseline: 2.6199x; 1.4137x over previous
import functools

import jax
import jax.numpy as jnp
from jax import lax
from jax.experimental import pallas as pl
from jax.experimental.pallas import tpu as pltpu
from jax.experimental.pallas import tpu_sc as plsc

N = 10000
R = 16
H = 256
C = 256
E = 160000

NC = 2
NS = 16
L = 16
EPS = E // NS
EPSC = E // (NC * NS)
W = 256
QW = W // 2
BB = 320
NBLK = -(-EPSC // BB)
PE = E + 512
ACC_ROWS = 10240
STRIPE = ACC_ROWS // NS

_mesh = plsc.VectorSubcoreMesh(core_axis_name="c", subcore_axis_name="s")


def _fill(ref, n, value):
    v = jnp.full((L,), value, ref.dtype)

    def body(i, _):
        ref[pl.ds(i * L, L)] = v
        return 0

    lax.fori_loop(0, n // L, body, 0)


def _en_body(rel_hbm, dst_hbm, en_hbm, rel_v, dst_v, comb_v, ones_v, cnt_v,
             hist_sh, sem):
    c = lax.axis_index("c")
    s = lax.axis_index("s")
    base = s * EPS
    _fill(cnt_v, EPS, 0.0)
    _fill(ones_v, EPS, 1.0)
    pltpu.sync_copy(cnt_v, hist_sh.at[pl.ds(s * (R * N // NS), R * N // NS)])
    pltpu.sync_copy(rel_hbm.at[pl.ds(base, EPS)], rel_v)
    pltpu.sync_copy(dst_hbm.at[pl.ds(base, EPS)], dst_v)

    def comb(i, _):
        sl = pl.ds(i * L, L)
        comb_v[sl] = rel_v[sl] * N + dst_v[sl]
        return 0

    lax.fori_loop(0, EPS // L, comb, 0)
    plsc.subcore_barrier()
    pltpu.sync_copy(ones_v, hist_sh.at[comb_v], add=True)
    plsc.subcore_barrier()

    wid = c * NS + s
    ebase = wid * EPSC
    pltpu.sync_copy(rel_hbm.at[pl.ds(ebase, EPSC)], rel_v.at[pl.ds(0, EPSC)])
    pltpu.sync_copy(dst_hbm.at[pl.ds(ebase, EPSC)], dst_v.at[pl.ds(0, EPSC)])

    def comb2(i, _):
        sl = pl.ds(i * L, L)
        comb_v[sl] = rel_v[sl] * N + dst_v[sl]
        return 0

    lax.fori_loop(0, -(-EPSC // L), comb2, 0)
    pltpu.async_copy(hist_sh.at[comb_v.at[pl.ds(0, EPSC)]],
                     cnt_v.at[pl.ds(0, EPSC)], sem).wait()

    def inv(i, _):
        sl = pl.ds(i * L, L)
        cnt_v[sl] = 1.0 / jnp.maximum(cnt_v[sl], 1.0)
        return 0

    lax.fori_loop(0, -(-EPSC // L), inv, 0)
    pltpu.sync_copy(cnt_v.at[pl.ds(0, EPSC)], en_hbm.at[pl.ds(ebase, EPSC)])


@functools.partial(
    pl.kernel,
    out_type=jax.ShapeDtypeStruct((E,), jnp.float32),
    mesh=_mesh,
    scratch_types=[
        pltpu.VMEM((EPS,), jnp.int32),
        pltpu.VMEM((EPS,), jnp.int32),
        pltpu.VMEM((EPS,), jnp.int32),
        pltpu.VMEM((EPS,), jnp.float32),
        pltpu.VMEM((EPS,), jnp.float32),
        pltpu.VMEM_SHARED((R * N,), jnp.float32),
        pltpu.SemaphoreType.DMA,
    ],
)
def _en_sc(rel_hbm, dst_hbm, *rest):
    _en_body(rel_hbm, dst_hbm, *rest)


def _gs_body(table_hbm, rel_hbm, src_hbm, dst_hbm, en_hbm, out_hbm,
             ga_v, src_v, sx_v, en_v, rows_v, acc_sh, sem):
    c = lax.axis_index("c")
    s = lax.axis_index("s")
    wid = c * NS + s
    ebase = wid * EPSC
    rows2 = rows_v
    lane = lax.iota(jnp.int32, L)

    for q in range(2):
        def zrow(i, _):
            for j in range(QW // L):
                rows2[i, pl.ds(j * L, L)] = jnp.zeros((L,), jnp.float32)
            return 0

        lax.fori_loop(0, BB, zrow, 0)
        for k0 in range(STRIPE // BB):
            pltpu.sync_copy(rows2,
                            acc_sh.at[pl.ds(s * STRIPE + k0 * BB, BB)])
        if STRIPE % BB:
            pltpu.sync_copy(
                rows2.at[pl.ds(0, STRIPE % BB)],
                acc_sh.at[pl.ds(s * STRIPE + (STRIPE // BB) * BB,
                                STRIPE % BB)])
        plsc.subcore_barrier()

        def block(b, _):
            base = ebase + b * BB
            pltpu.sync_copy(rel_hbm.at[pl.ds(base, BB)], ga_v)
            pltpu.sync_copy(src_hbm.at[pl.ds(base, BB)], src_v)
            pltpu.sync_copy(dst_hbm.at[pl.ds(base, BB)], sx_v)
            pltpu.sync_copy(en_hbm.at[pl.ds(base, BB)], en_v)

            def prep(i, _):
                sl = pl.ds(i * L, L)
                ga_v[sl] = (ga_v[sl] * N + src_v[sl]) * 2 + q
                pos = b * BB + i * L + lane
                en_v[sl] = jnp.where(pos < EPSC, en_v[sl], 0.0)
                return 0

            lax.fori_loop(0, BB // L, prep, 0)
            pltpu.async_copy(table_hbm.at[ga_v], rows2, sem).wait()

            def scale(g, _):
                ev16 = en_v[pl.ds(g * L, L)]
                for r in range(L):
                    ev = lax.broadcast_in_dim(ev16[r], (L,), ())
                    i = g * L + r
                    for j in range(QW // L):
                        sl = pl.ds(j * L, L)
                        rows2[i, sl] = rows2[i, sl] * ev
                return 0

            lax.fori_loop(0, BB // L, scale, 0)
            pltpu.sync_copy(rows2, acc_sh.at[sx_v], add=True)
            return 0

        lax.fori_loop(0, NBLK, block, 0)
        plsc.subcore_barrier()

        @pl.when(s < NS - 1)
        def _():
            pltpu.sync_copy(
                acc_sh.at[pl.ds(s * STRIPE, STRIPE)],
                out_hbm.at[pl.ds(c * N + s * STRIPE, STRIPE),
                           pl.ds(q * QW, QW)])

        @pl.when(s == NS - 1)
        def _():
            last = N - (NS - 1) * STRIPE
            pltpu.sync_copy(
                acc_sh.at[pl.ds((NS - 1) * STRIPE, last)],
                out_hbm.at[pl.ds(c * N + (NS - 1) * STRIPE, last),
                           pl.ds(q * QW, QW)])

        plsc.subcore_barrier()


@functools.partial(
    pl.kernel,
    out_type=jax.ShapeDtypeStruct((NC * N, W), jnp.float32),
    mesh=_mesh,
    scratch_types=[
        pltpu.VMEM((BB,), jnp.int32),
        pltpu.VMEM((BB,), jnp.int32),
        pltpu.VMEM((BB,), jnp.int32),
        pltpu.VMEM((BB,), jnp.float32),
        pltpu.VMEM((BB, QW), jnp.float32),
        pltpu.VMEM_SHARED((ACC_ROWS, QW), jnp.float32),
        pltpu.SemaphoreType.DMA,
    ],
)
def _gs_sc(table_hbm, rel_hbm, src_hbm, dst_hbm, en_hbm, *rest):
    _gs_body(table_hbm, rel_hbm, src_hbm, dst_hbm, en_hbm, *rest)


BLK = 400


def _h_block(p1_ref, root1_ref, bias1_ref):
    h = p1_ref[0] + p1_ref[1] + root1_ref[...] + bias1_ref[...]
    return jnp.maximum(h, 0.0).astype(jnp.bfloat16)


def _y_block(p1_ref, root1_ref, bias1_ref, w2_ref, y_ref):
    h = _h_block(p1_ref, root1_ref, bias1_ref)

    def body(r, _):
        y_ref[r] = jnp.dot(h, w2_ref[r], preferred_element_type=jnp.float32)
        return 0

    lax.fori_loop(0, R, body, 0, unroll=True)


def _y_tc(p1, root1, bias1, w2_bf16):
    return pl.pallas_call(
        _y_block,
        grid=(N // BLK,),
        in_specs=[
            pl.BlockSpec((NC, BLK, H), lambda i: (0, i, 0)),
            pl.BlockSpec((BLK, H), lambda i: (i, 0)),
            pl.BlockSpec((1, H), lambda i: (0, 0)),
            pl.BlockSpec((R, H, C), lambda i: (0, 0, 0)),
        ],
        out_specs=pl.BlockSpec((R, BLK, C), lambda i: (0, i, 0)),
        out_shape=jax.ShapeDtypeStruct((R, N, C), jnp.float32),
    )(p1, root1, bias1, w2_bf16)


def _tail_block(p2_ref, p1_ref, root1_ref, bias1_ref, root2_ref,
                bias2_ref, out_ref):
    h = _h_block(p1_ref, root1_ref, bias1_ref)
    o = p2_ref[0] + p2_ref[1] + jnp.dot(h, root2_ref[...],
                                        preferred_element_type=jnp.float32)
    o = o + bias2_ref[...]
    m = jnp.max(o, axis=1, keepdims=True)
    lse = jnp.log(jnp.sum(jnp.exp(o - m), axis=1, keepdims=True))
    out_ref[...] = o - m - lse


def _tail_tc(p2, p1, root1, bias1, root2_bf16, bias2):
    return pl.pallas_call(
        _tail_block,
        grid=(N // BLK,),
        in_specs=[
            pl.BlockSpec((NC, BLK, C), lambda i: (0, i, 0)),
            pl.BlockSpec((NC, BLK, H), lambda i: (0, i, 0)),
            pl.BlockSpec((BLK, H), lambda i: (i, 0)),
            pl.BlockSpec((1, H), lambda i: (0, 0)),
            pl.BlockSpec((H, C), lambda i: (0, 0)),
            pl.BlockSpec((1, C), lambda i: (0, 0)),
        ],
        out_specs=pl.BlockSpec((BLK, C), lambda i: (i, 0)),
        out_shape=jax.ShapeDtypeStruct((N, C), jnp.float32),
    )(p2, p1, root1, bias1, root2_bf16, bias2)


def kernel(edge_index, edge_attr, weight1, root1, bias1, weight2, root2, bias2):
    src = edge_index[0].astype(jnp.int32)
    dst = edge_index[1].astype(jnp.int32)
    rel = edge_attr.astype(jnp.int32)
    en = _en_sc(rel, dst)
    pad = PE - E
    rel = jnp.pad(rel, (0, pad))
    src = jnp.pad(src, (0, pad))
    dst = jnp.pad(dst, (0, pad))
    en = jnp.pad(en, (0, pad))
    p1 = _gs_sc(weight1.reshape(R * N * 2, QW), rel, src, dst,
                en).reshape(NC, N, W)
    y = _y_tc(p1, root1, bias1.reshape(1, H), weight2.astype(jnp.bfloat16))
    p2 = _gs_sc(y.reshape(R * N * 2, QW), rel, src, dst,
                en).reshape(NC, N, C)
    return _tail_tc(p2, p1, root1, bias1.reshape(1, H),
                    root2.astype(jnp.bfloat16), bias2.reshape(1, C))


# speedup vs baseline: 2.7145x; 1.0361x over previous
import functools

import jax
import jax.numpy as jnp
from jax import lax
from jax.experimental import pallas as pl
from jax.experimental.pallas import tpu as pltpu
from jax.experimental.pallas import tpu_sc as plsc

N = 10000
R = 16
H = 256
C = 256
E = 160000

NC = 2
NS = 16
L = 16
EPS = E // NS
EPSC = E // (NC * NS)
W = 256
QW = W // 2
BB = 160
NBLK = -(-EPSC // BB)
assert NBLK % 2 == 0
PE = E + 512
ACC_ROWS = 10240
STRIPE = ACC_ROWS // NS

_mesh = plsc.VectorSubcoreMesh(core_axis_name="c", subcore_axis_name="s")


def _fill(ref, n, value):
    v = jnp.full((L,), value, ref.dtype)

    def body(i, _):
        ref[pl.ds(i * L, L)] = v
        return 0

    lax.fori_loop(0, n // L, body, 0)


def _en_body(rel_hbm, dst_hbm, en_hbm, rel_v, dst_v, comb_v, ones_v, cnt_v,
             hist_sh, sem):
    c = lax.axis_index("c")
    s = lax.axis_index("s")
    base = s * EPS
    _fill(cnt_v, EPS, 0.0)
    _fill(ones_v, EPS, 1.0)
    pltpu.sync_copy(cnt_v, hist_sh.at[pl.ds(s * (R * N // NS), R * N // NS)])
    pltpu.sync_copy(rel_hbm.at[pl.ds(base, EPS)], rel_v)
    pltpu.sync_copy(dst_hbm.at[pl.ds(base, EPS)], dst_v)

    def comb(i, _):
        sl = pl.ds(i * L, L)
        comb_v[sl] = rel_v[sl] * N + dst_v[sl]
        return 0

    lax.fori_loop(0, EPS // L, comb, 0)
    plsc.subcore_barrier()
    pltpu.sync_copy(ones_v, hist_sh.at[comb_v], add=True)
    plsc.subcore_barrier()

    wid = c * NS + s
    ebase = wid * EPSC
    pltpu.sync_copy(rel_hbm.at[pl.ds(ebase, EPSC)], rel_v.at[pl.ds(0, EPSC)])
    pltpu.sync_copy(dst_hbm.at[pl.ds(ebase, EPSC)], dst_v.at[pl.ds(0, EPSC)])

    def comb2(i, _):
        sl = pl.ds(i * L, L)
        comb_v[sl] = rel_v[sl] * N + dst_v[sl]
        return 0

    lax.fori_loop(0, -(-EPSC // L), comb2, 0)
    pltpu.async_copy(hist_sh.at[comb_v.at[pl.ds(0, EPSC)]],
                     cnt_v.at[pl.ds(0, EPSC)], sem).wait()

    def inv(i, _):
        sl = pl.ds(i * L, L)
        cnt_v[sl] = 1.0 / jnp.maximum(cnt_v[sl], 1.0)
        return 0

    lax.fori_loop(0, -(-EPSC // L), inv, 0)
    pltpu.sync_copy(cnt_v.at[pl.ds(0, EPSC)], en_hbm.at[pl.ds(ebase, EPSC)])


@functools.partial(
    pl.kernel,
    out_type=jax.ShapeDtypeStruct((E,), jnp.float32),
    mesh=_mesh,
    scratch_types=[
        pltpu.VMEM((EPS,), jnp.int32),
        pltpu.VMEM((EPS,), jnp.int32),
        pltpu.VMEM((EPS,), jnp.int32),
        pltpu.VMEM((EPS,), jnp.float32),
        pltpu.VMEM((EPS,), jnp.float32),
        pltpu.VMEM_SHARED((R * N,), jnp.float32),
        pltpu.SemaphoreType.DMA,
    ],
)
def _en_sc(rel_hbm, dst_hbm, *rest):
    _en_body(rel_hbm, dst_hbm, *rest)


def _gs_body(table_hbm, eidx_hbm, en_hbm, out_hbm,
             ga_a, ga_b, sx_a, sx_b, en_a, en_b, src_v, rows_a, rows_b,
             acc_sh, sem_a, sem_b):
    c = lax.axis_index("c")
    s = lax.axis_index("s")
    wid = c * NS + s
    ebase = wid * EPSC
    lane = lax.iota(jnp.int32, L)

    def stage(b, q, ga_v, sx_v, en_v):
        base = ebase + b * BB
        pltpu.sync_copy(eidx_hbm.at[pl.ds(base, BB)], ga_v)
        pltpu.sync_copy(eidx_hbm.at[pl.ds(PE + base, BB)], src_v)
        pltpu.sync_copy(eidx_hbm.at[pl.ds(2 * PE + base, BB)], sx_v)
        pltpu.sync_copy(en_hbm.at[pl.ds(base, BB)], en_v)

        def prep(i, _):
            sl = pl.ds(i * L, L)
            ga_v[sl] = (ga_v[sl] * N + src_v[sl]) * 2 + q
            pos = b * BB + i * L + lane
            en_v[sl] = jnp.where(pos < EPSC, en_v[sl], 0.0)
            return 0

        lax.fori_loop(0, BB // L, prep, 0)

    def process(rows2, sx_v, en_v):
        def scale(g, _):
            ev16 = en_v[pl.ds(g * L, L)]
            for r in range(L):
                ev = lax.broadcast_in_dim(ev16[r], (L,), ())
                i = g * L + r
                for j in range(QW // L):
                    sl = pl.ds(j * L, L)
                    rows2[i, sl] = rows2[i, sl] * ev
            return 0

        lax.fori_loop(0, BB // L, scale, 0)
        pltpu.sync_copy(rows2, acc_sh.at[sx_v], add=True)

    for q in range(2):
        def zrow(i, _):
            for j in range(QW // L):
                rows_a[i, pl.ds(j * L, L)] = jnp.zeros((L,), jnp.float32)
            return 0

        lax.fori_loop(0, BB, zrow, 0)
        for k0 in range(-(-STRIPE // BB)):
            nrow = min(BB, STRIPE - k0 * BB)
            pltpu.sync_copy(rows_a.at[pl.ds(0, nrow)],
                            acc_sh.at[pl.ds(s * STRIPE + k0 * BB, nrow)])
        plsc.subcore_barrier()

        stage(0, q, ga_a, sx_a, en_a)
        pltpu.async_copy(table_hbm.at[ga_a], rows_a, sem_a)

        def pair(k, _):
            b = 2 * k
            stage(b + 1, q, ga_b, sx_b, en_b)
            pltpu.async_copy(table_hbm.at[ga_b], rows_b, sem_b)
            pltpu.make_async_copy(table_hbm.at[ga_a], rows_a, sem_a).wait()
            process(rows_a, sx_a, en_a)

            @pl.when(k < NBLK // 2 - 1)
            def _():
                stage(b + 2, q, ga_a, sx_a, en_a)
                pltpu.async_copy(table_hbm.at[ga_a], rows_a, sem_a)

            pltpu.make_async_copy(table_hbm.at[ga_b], rows_b, sem_b).wait()
            process(rows_b, sx_b, en_b)
            return 0

        lax.fori_loop(0, NBLK // 2, pair, 0)
        plsc.subcore_barrier()

        @pl.when(s < NS - 1)
        def _():
            pltpu.sync_copy(
                acc_sh.at[pl.ds(s * STRIPE, STRIPE)],
                out_hbm.at[pl.ds(c * N + s * STRIPE, STRIPE),
                           pl.ds(q * QW, QW)])

        @pl.when(s == NS - 1)
        def _():
            last = N - (NS - 1) * STRIPE
            pltpu.sync_copy(
                acc_sh.at[pl.ds((NS - 1) * STRIPE, last)],
                out_hbm.at[pl.ds(c * N + (NS - 1) * STRIPE, last),
                           pl.ds(q * QW, QW)])

        plsc.subcore_barrier()


@functools.partial(
    pl.kernel,
    out_type=jax.ShapeDtypeStruct((NC * N, W), jnp.float32),
    mesh=_mesh,
    scratch_types=[
        pltpu.VMEM((BB,), jnp.int32),
        pltpu.VMEM((BB,), jnp.int32),
        pltpu.VMEM((BB,), jnp.int32),
        pltpu.VMEM((BB,), jnp.int32),
        pltpu.VMEM((BB,), jnp.float32),
        pltpu.VMEM((BB,), jnp.float32),
        pltpu.VMEM((BB,), jnp.int32),
        pltpu.VMEM((BB, QW), jnp.float32),
        pltpu.VMEM((BB, QW), jnp.float32),
        pltpu.VMEM_SHARED((ACC_ROWS, QW), jnp.float32),
        pltpu.SemaphoreType.DMA,
        pltpu.SemaphoreType.DMA,
    ],
)
def _gs_sc(table_hbm, eidx_hbm, en_hbm, *rest):
    _gs_body(table_hbm, eidx_hbm, en_hbm, *rest)


BLK = 400


def _h_block(p1_ref, root1_ref, bias1_ref):
    h = p1_ref[0] + p1_ref[1] + root1_ref[...] + bias1_ref[...]
    return jnp.maximum(h, 0.0).astype(jnp.bfloat16)


def _y_block(p1_ref, root1_ref, bias1_ref, w2_ref, y_ref):
    h = _h_block(p1_ref, root1_ref, bias1_ref)

    def body(r, _):
        y_ref[r] = jnp.dot(h, w2_ref[r], preferred_element_type=jnp.float32)
        return 0

    lax.fori_loop(0, R, body, 0, unroll=True)


def _y_tc(p1, root1, bias1, w2_bf16):
    return pl.pallas_call(
        _y_block,
        grid=(N // BLK,),
        in_specs=[
            pl.BlockSpec((NC, BLK, H), lambda i: (0, i, 0)),
            pl.BlockSpec((BLK, H), lambda i: (i, 0)),
            pl.BlockSpec((1, H), lambda i: (0, 0)),
            pl.BlockSpec((R, H, C), lambda i: (0, 0, 0)),
        ],
        out_specs=pl.BlockSpec((R, BLK, C), lambda i: (0, i, 0)),
        out_shape=jax.ShapeDtypeStruct((R, N, C), jnp.float32),
    )(p1, root1, bias1, w2_bf16)


def _tail_block(p2_ref, p1_ref, root1_ref, bias1_ref, root2_ref,
                bias2_ref, out_ref):
    h = _h_block(p1_ref, root1_ref, bias1_ref)
    o = p2_ref[0] + p2_ref[1] + jnp.dot(h, root2_ref[...],
                                        preferred_element_type=jnp.float32)
    o = o + bias2_ref[...]
    m = jnp.max(o, axis=1, keepdims=True)
    lse = jnp.log(jnp.sum(jnp.exp(o - m), axis=1, keepdims=True))
    out_ref[...] = o - m - lse


def _tail_tc(p2, p1, root1, bias1, root2_bf16, bias2):
    return pl.pallas_call(
        _tail_block,
        grid=(N // BLK,),
        in_specs=[
            pl.BlockSpec((NC, BLK, C), lambda i: (0, i, 0)),
            pl.BlockSpec((NC, BLK, H), lambda i: (0, i, 0)),
            pl.BlockSpec((BLK, H), lambda i: (i, 0)),
            pl.BlockSpec((1, H), lambda i: (0, 0)),
            pl.BlockSpec((H, C), lambda i: (0, 0)),
            pl.BlockSpec((1, C), lambda i: (0, 0)),
        ],
        out_specs=pl.BlockSpec((BLK, C), lambda i: (i, 0)),
        out_shape=jax.ShapeDtypeStruct((N, C), jnp.float32),
    )(p2, p1, root1, bias1, root2_bf16, bias2)


def kernel(edge_index, edge_attr, weight1, root1, bias1, weight2, root2, bias2):
    src = edge_index[0].astype(jnp.int32)
    dst = edge_index[1].astype(jnp.int32)
    rel = edge_attr.astype(jnp.int32)
    en = _en_sc(rel, dst)
    eidx = jnp.stack([rel, src, dst])
    eidx = jnp.pad(eidx, ((0, 0), (0, PE - E))).reshape(3 * PE)

    enp = jnp.pad(en, (0, PE - E))
    p1 = _gs_sc(weight1.reshape(R * N * 2, QW), eidx, enp).reshape(NC, N, W)
    y = _y_tc(p1, root1, bias1.reshape(1, H), weight2.astype(jnp.bfloat16))
    p2 = _gs_sc(y.reshape(R * N * 2, QW), eidx, enp).reshape(NC, N, C)
    return _tail_tc(p2, p1, root1, bias1.reshape(1, H),
                    root2.astype(jnp.bfloat16), bias2.reshape(1, C))


# speedup vs baseline: 3.2814x; 1.2088x over previous
import functools

import jax
import jax.numpy as jnp
from jax import lax
from jax.experimental import pallas as pl
from jax.experimental.pallas import tpu as pltpu
from jax.experimental.pallas import tpu_sc as plsc

N = 10000
R = 16
H = 256
C = 256
E = 160000

NC = 2
NS = 16
L = 16
EPS = E // NS
EPSC = E // (NC * NS)
W = 256
QW = W // 2
BB = 96
NBLK = 54
BIGE = NBLK * BB
PE = E + BIGE - EPSC + 64
ACC_ROWS = 10240
STRIPE = ACC_ROWS // NS

_mesh = plsc.VectorSubcoreMesh(core_axis_name="c", subcore_axis_name="s")


def _fill(ref, n, value):
    v = jnp.full((L,), value, ref.dtype)

    def body(i, _):
        ref[pl.ds(i * L, L)] = v
        return 0

    lax.fori_loop(0, n // L, body, 0)


def _en_body(rel_hbm, dst_hbm, en_hbm, rel_v, dst_v, comb_v, ones_v, cnt_v,
             hist_sh, sem):
    c = lax.axis_index("c")
    s = lax.axis_index("s")
    base = s * EPS
    _fill(cnt_v, EPS, 0.0)
    _fill(ones_v, EPS, 1.0)
    pltpu.sync_copy(cnt_v, hist_sh.at[pl.ds(s * (R * N // NS), R * N // NS)])
    pltpu.sync_copy(rel_hbm.at[pl.ds(base, EPS)], rel_v)
    pltpu.sync_copy(dst_hbm.at[pl.ds(base, EPS)], dst_v)

    def comb(i, _):
        sl = pl.ds(i * L, L)
        comb_v[sl] = rel_v[sl] * N + dst_v[sl]
        return 0

    lax.fori_loop(0, EPS // L, comb, 0)
    plsc.subcore_barrier()
    pltpu.sync_copy(ones_v, hist_sh.at[comb_v], add=True)
    plsc.subcore_barrier()

    wid = c * NS + s
    ebase = wid * EPSC
    pltpu.sync_copy(rel_hbm.at[pl.ds(ebase, EPSC)], rel_v.at[pl.ds(0, EPSC)])
    pltpu.sync_copy(dst_hbm.at[pl.ds(ebase, EPSC)], dst_v.at[pl.ds(0, EPSC)])

    def comb2(i, _):
        sl = pl.ds(i * L, L)
        comb_v[sl] = rel_v[sl] * N + dst_v[sl]
        return 0

    lax.fori_loop(0, -(-EPSC // L), comb2, 0)
    pltpu.async_copy(hist_sh.at[comb_v.at[pl.ds(0, EPSC)]],
                     cnt_v.at[pl.ds(0, EPSC)], sem).wait()

    def inv(i, _):
        sl = pl.ds(i * L, L)
        cnt_v[sl] = 1.0 / jnp.maximum(cnt_v[sl], 1.0)
        return 0

    lax.fori_loop(0, -(-EPSC // L), inv, 0)
    pltpu.sync_copy(cnt_v.at[pl.ds(0, EPSC)], en_hbm.at[pl.ds(ebase, EPSC)])


@functools.partial(
    pl.kernel,
    out_type=jax.ShapeDtypeStruct((E,), jnp.float32),
    mesh=_mesh,
    scratch_types=[
        pltpu.VMEM((EPS,), jnp.int32),
        pltpu.VMEM((EPS,), jnp.int32),
        pltpu.VMEM((EPS,), jnp.int32),
        pltpu.VMEM((EPS,), jnp.float32),
        pltpu.VMEM((EPS,), jnp.float32),
        pltpu.VMEM_SHARED((R * N,), jnp.float32),
        pltpu.SemaphoreType.DMA,
    ],
)
def _en_sc(rel_hbm, dst_hbm, *rest):
    _en_body(rel_hbm, dst_hbm, *rest)


def _gs_body(table_hbm, eidx_hbm, en_hbm, out_hbm,
             idx0_v, idx1_v, dst_v, en_v, sx_a, sx_b, rows_a, rows_b,
             acc_sh, sem_a, sem_b):
    c = lax.axis_index("c")
    s = lax.axis_index("s")
    wid = c * NS + s
    ebase = wid * EPSC
    lane = lax.iota(jnp.int32, L)

    pltpu.sync_copy(eidx_hbm.at[pl.ds(ebase, BIGE)], idx0_v)
    pltpu.sync_copy(eidx_hbm.at[pl.ds(PE + ebase, BIGE)], idx1_v)
    pltpu.sync_copy(eidx_hbm.at[pl.ds(2 * PE + ebase, BIGE)], dst_v)
    pltpu.sync_copy(en_hbm.at[pl.ds(ebase, BIGE)], en_v)

    def prep(i, _):
        sl = pl.ds(i * L, L)
        g2 = (idx0_v[sl] * N + idx1_v[sl]) * 2
        idx0_v[sl] = g2
        idx1_v[sl] = g2 + 1
        pos = i * L + lane
        en_v[sl] = jnp.where(pos < EPSC, en_v[sl], 0.0)
        return 0

    lax.fori_loop(0, BIGE // L, prep, 0)

    def fire(b, idx_q, rows2, sem):
        pltpu.async_copy(table_hbm.at[idx_q.at[pl.ds(b * BB, BB)]],
                         rows2, sem)

    def stage_sx(b, sx_v):
        def cp(i, _):
            sl = pl.ds(i * L, L)
            sx_v[sl] = dst_v[pl.ds(b * BB + i * L, L)]
            return 0

        lax.fori_loop(0, BB // L, cp, 0)

    def process(b, rows2, sx_v):
        boff = b * BB

        def scale(g, _):
            ev16 = en_v[pl.ds(boff + g * L, L)]
            for r in range(L):
                ev = lax.broadcast_in_dim(ev16[r], (L,), ())
                i = g * L + r
                for j in range(QW // L):
                    sl = pl.ds(j * L, L)
                    rows2[i, sl] = rows2[i, sl] * ev
            return 0

        lax.fori_loop(0, BB // L, scale, 0)
        pltpu.sync_copy(rows2, acc_sh.at[sx_v], add=True)

    for q in range(2):
        idx_q = idx0_v if q == 0 else idx1_v

        def zrow(i, _):
            for j in range(QW // L):
                rows_a[i, pl.ds(j * L, L)] = jnp.zeros((L,), jnp.float32)
            return 0

        lax.fori_loop(0, BB, zrow, 0)
        for k0 in range(-(-STRIPE // BB)):
            nrow = min(BB, STRIPE - k0 * BB)
            pltpu.sync_copy(rows_a.at[pl.ds(0, nrow)],
                            acc_sh.at[pl.ds(s * STRIPE + k0 * BB, nrow)])
        plsc.subcore_barrier()

        stage_sx(0, sx_a)
        fire(0, idx_q, rows_a, sem_a)

        def pair(k, _):
            b = 2 * k
            stage_sx(b + 1, sx_b)
            fire(b + 1, idx_q, rows_b, sem_b)
            pltpu.make_async_copy(table_hbm.at[idx_q.at[pl.ds(0, BB)]],
                                  rows_a, sem_a).wait()
            process(b, rows_a, sx_a)

            @pl.when(k < NBLK // 2 - 1)
            def _():
                stage_sx(b + 2, sx_a)
                fire(b + 2, idx_q, rows_a, sem_a)

            pltpu.make_async_copy(table_hbm.at[idx_q.at[pl.ds(0, BB)]],
                                  rows_b, sem_b).wait()
            process(b + 1, rows_b, sx_b)
            return 0

        lax.fori_loop(0, NBLK // 2, pair, 0)
        plsc.subcore_barrier()

        @pl.when(s < NS - 1)
        def _():
            pltpu.sync_copy(
                acc_sh.at[pl.ds(s * STRIPE, STRIPE)],
                out_hbm.at[pl.ds(c * N + s * STRIPE, STRIPE),
                           pl.ds(q * QW, QW)])

        @pl.when(s == NS - 1)
        def _():
            last = N - (NS - 1) * STRIPE
            pltpu.sync_copy(
                acc_sh.at[pl.ds((NS - 1) * STRIPE, last)],
                out_hbm.at[pl.ds(c * N + (NS - 1) * STRIPE, last),
                           pl.ds(q * QW, QW)])

        plsc.subcore_barrier()


@functools.partial(
    pl.kernel,
    out_type=jax.ShapeDtypeStruct((NC * N, W), jnp.float32),
    mesh=_mesh,
    scratch_types=[
        pltpu.VMEM((BIGE,), jnp.int32),
        pltpu.VMEM((BIGE,), jnp.int32),
        pltpu.VMEM((BIGE,), jnp.int32),
        pltpu.VMEM((BIGE,), jnp.float32),
        pltpu.VMEM((BB,), jnp.int32),
        pltpu.VMEM((BB,), jnp.int32),
        pltpu.VMEM((BB, QW), jnp.float32),
        pltpu.VMEM((BB, QW), jnp.float32),
        pltpu.VMEM_SHARED((ACC_ROWS, QW), jnp.float32),
        pltpu.SemaphoreType.DMA,
        pltpu.SemaphoreType.DMA,
    ],
)
def _gs_sc(table_hbm, eidx_hbm, en_hbm, *rest):
    _gs_body(table_hbm, eidx_hbm, en_hbm, *rest)


BLK = 400


def _h_block(p1_ref, root1_ref, bias1_ref):
    h = p1_ref[0] + p1_ref[1] + root1_ref[...] + bias1_ref[...]
    return jnp.maximum(h, 0.0).astype(jnp.bfloat16)


def _y_block(p1_ref, root1_ref, bias1_ref, w2_ref, y_ref):
    h = _h_block(p1_ref, root1_ref, bias1_ref)

    def body(r, _):
        y_ref[r] = jnp.dot(h, w2_ref[r], preferred_element_type=jnp.float32)
        return 0

    lax.fori_loop(0, R, body, 0, unroll=True)


def _y_tc(p1, root1, bias1, w2_bf16):
    return pl.pallas_call(
        _y_block,
        grid=(N // BLK,),
        in_specs=[
            pl.BlockSpec((NC, BLK, H), lambda i: (0, i, 0)),
            pl.BlockSpec((BLK, H), lambda i: (i, 0)),
            pl.BlockSpec((1, H), lambda i: (0, 0)),
            pl.BlockSpec((R, H, C), lambda i: (0, 0, 0)),
        ],
        out_specs=pl.BlockSpec((R, BLK, C), lambda i: (0, i, 0)),
        out_shape=jax.ShapeDtypeStruct((R, N, C), jnp.float32),
    )(p1, root1, bias1, w2_bf16)


def _tail_block(p2_ref, p1_ref, root1_ref, bias1_ref, root2_ref,
                bias2_ref, out_ref):
    h = _h_block(p1_ref, root1_ref, bias1_ref)
    o = p2_ref[0] + p2_ref[1] + jnp.dot(h, root2_ref[...],
                                        preferred_element_type=jnp.float32)
    o = o + bias2_ref[...]
    m = jnp.max(o, axis=1, keepdims=True)
    lse = jnp.log(jnp.sum(jnp.exp(o - m), axis=1, keepdims=True))
    out_ref[...] = o - m - lse


def _tail_tc(p2, p1, root1, bias1, root2_bf16, bias2):
    return pl.pallas_call(
        _tail_block,
        grid=(N // BLK,),
        in_specs=[
            pl.BlockSpec((NC, BLK, C), lambda i: (0, i, 0)),
            pl.BlockSpec((NC, BLK, H), lambda i: (0, i, 0)),
            pl.BlockSpec((BLK, H), lambda i: (i, 0)),
            pl.BlockSpec((1, H), lambda i: (0, 0)),
            pl.BlockSpec((H, C), lambda i: (0, 0)),
            pl.BlockSpec((1, C), lambda i: (0, 0)),
        ],
        out_specs=pl.BlockSpec((BLK, C), lambda i: (i, 0)),
        out_shape=jax.ShapeDtypeStruct((N, C), jnp.float32),
    )(p2, p1, root1, bias1, root2_bf16, bias2)


def kernel(edge_index, edge_attr, weight1, root1, bias1, weight2, root2, bias2):
    src = edge_index[0].astype(jnp.int32)
    dst = edge_index[1].astype(jnp.int32)
    rel = edge_attr.astype(jnp.int32)
    en = _en_sc(rel, dst)
    eidx = jnp.stack([rel, src, dst])
    eidx = jnp.pad(eidx, ((0, 0), (0, PE - E))).reshape(3 * PE)

    enp = jnp.pad(en, (0, PE - E))
    p1 = _gs_sc(weight1.reshape(R * N * 2, QW), eidx, enp).reshape(NC, N, W)
    y = _y_tc(p1, root1, bias1.reshape(1, H), weight2.astype(jnp.bfloat16))
    p2 = _gs_sc(y.reshape(R * N * 2, QW), eidx, enp).reshape(NC, N, C)
    return _tail_tc(p2, p1, root1, bias1.reshape(1, H),
                    root2.astype(jnp.bfloat16), bias2.reshape(1, C))


# speedup vs baseline: 4.1454x; 1.2633x over previous
import functools

import jax
import jax.numpy as jnp
from jax import lax
from jax.experimental import pallas as pl
from jax.experimental.pallas import tpu as pltpu
from jax.experimental.pallas import tpu_sc as plsc

N = 10000
R = 16
H = 256
C = 256
E = 160000

NC = 2
NS = 16
L = 16
EPS = E // NS
EPSC = E // (NC * NS)
W = 256
QW = W // 2
BB = 96
NBLK = 54
BIGE = NBLK * BB
PE = E + BIGE - EPSC + 64
ACC_ROWS = 10240
STRIPE = ACC_ROWS // NS

_mesh = plsc.VectorSubcoreMesh(core_axis_name="c", subcore_axis_name="s")


def _fill(ref, n, value):
    v = jnp.full((L,), value, ref.dtype)

    def body(i, _):
        ref[pl.ds(i * L, L)] = v
        return 0

    lax.fori_loop(0, n // L, body, 0)


def _en_body(rel_hbm, dst_hbm, en_hbm, rel_v, dst_v, comb_v, ones_v, cnt_v,
             hist_sh, sem):
    c = lax.axis_index("c")
    s = lax.axis_index("s")
    base = s * EPS
    _fill(cnt_v, EPS, 0.0)
    _fill(ones_v, EPS, 1.0)
    pltpu.sync_copy(cnt_v, hist_sh.at[pl.ds(s * (R * N // NS), R * N // NS)])
    pltpu.sync_copy(rel_hbm.at[pl.ds(base, EPS)], rel_v)
    pltpu.sync_copy(dst_hbm.at[pl.ds(base, EPS)], dst_v)

    def comb(i, _):
        sl = pl.ds(i * L, L)
        comb_v[sl] = rel_v[sl] * N + dst_v[sl]
        return 0

    lax.fori_loop(0, EPS // L, comb, 0)
    plsc.subcore_barrier()
    pltpu.sync_copy(ones_v, hist_sh.at[comb_v], add=True)
    plsc.subcore_barrier()

    wid = c * NS + s
    ebase = wid * EPSC
    pltpu.sync_copy(rel_hbm.at[pl.ds(ebase, EPSC)], rel_v.at[pl.ds(0, EPSC)])
    pltpu.sync_copy(dst_hbm.at[pl.ds(ebase, EPSC)], dst_v.at[pl.ds(0, EPSC)])

    def comb2(i, _):
        sl = pl.ds(i * L, L)
        comb_v[sl] = rel_v[sl] * N + dst_v[sl]
        return 0

    lax.fori_loop(0, -(-EPSC // L), comb2, 0)
    pltpu.async_copy(hist_sh.at[comb_v.at[pl.ds(0, EPSC)]],
                     cnt_v.at[pl.ds(0, EPSC)], sem).wait()

    def inv(i, _):
        sl = pl.ds(i * L, L)
        cnt_v[sl] = 1.0 / jnp.maximum(cnt_v[sl], 1.0)
        return 0

    lax.fori_loop(0, -(-EPSC // L), inv, 0)
    pltpu.sync_copy(cnt_v.at[pl.ds(0, EPSC)], en_hbm.at[pl.ds(ebase, EPSC)])


@functools.partial(
    pl.kernel,
    out_type=jax.ShapeDtypeStruct((E,), jnp.float32),
    mesh=_mesh,
    scratch_types=[
        pltpu.VMEM((EPS,), jnp.int32),
        pltpu.VMEM((EPS,), jnp.int32),
        pltpu.VMEM((EPS,), jnp.int32),
        pltpu.VMEM((EPS,), jnp.float32),
        pltpu.VMEM((EPS,), jnp.float32),
        pltpu.VMEM_SHARED((R * N,), jnp.float32),
        pltpu.SemaphoreType.DMA,
    ],
)
def _en_sc(rel_hbm, dst_hbm, *rest):
    _en_body(rel_hbm, dst_hbm, *rest)


def _gs_body(bm, qstep, table_hbm, eidx_hbm, en_hbm, out_hbm,
             idx0_v, idx1_v, dst_v, en_v, sx_a, sx_b, rows_a, rows_b,
             acc_sh, sem_a, sem_b):
    c = lax.axis_index("c")
    s = lax.axis_index("s")
    wid = c * NS + s
    ebase = wid * EPSC
    lane = lax.iota(jnp.int32, L)

    pltpu.sync_copy(eidx_hbm.at[pl.ds(ebase, BIGE)], idx0_v)
    pltpu.sync_copy(eidx_hbm.at[pl.ds(PE + ebase, BIGE)], idx1_v)
    pltpu.sync_copy(eidx_hbm.at[pl.ds(2 * PE + ebase, BIGE)], dst_v)
    pltpu.sync_copy(en_hbm.at[pl.ds(ebase, BIGE)], en_v)

    def prep(i, _):
        sl = pl.ds(i * L, L)
        g0 = idx0_v[sl] * (2 * N) + idx1_v[sl] * bm
        idx0_v[sl] = g0
        idx1_v[sl] = g0 + qstep
        pos = i * L + lane
        en_v[sl] = jnp.where(pos < EPSC, en_v[sl], 0.0)
        return 0

    lax.fori_loop(0, BIGE // L, prep, 0)

    def fire(b, idx_q, rows2, sem):
        pltpu.async_copy(table_hbm.at[idx_q.at[pl.ds(b * BB, BB)]],
                         rows2, sem)

    def stage_sx(b, sx_v):
        def cp(i, _):
            sl = pl.ds(i * L, L)
            sx_v[sl] = dst_v[pl.ds(b * BB + i * L, L)]
            return 0

        lax.fori_loop(0, BB // L, cp, 0)

    def process(b, rows2, sx_v):
        boff = b * BB

        def scale(g, _):
            ev16 = en_v[pl.ds(boff + g * L, L)]
            for r in range(L):
                ev = lax.broadcast_in_dim(ev16[r], (L,), ())
                i = g * L + r
                for j in range(QW // L):
                    sl = pl.ds(j * L, L)
                    rows2[i, sl] = rows2[i, sl] * ev
            return 0

        lax.fori_loop(0, BB // L, scale, 0)
        pltpu.sync_copy(rows2, acc_sh.at[sx_v], add=True)

    for q in range(2):
        idx_q = idx0_v if q == 0 else idx1_v

        def zrow(i, _):
            for j in range(QW // L):
                rows_a[i, pl.ds(j * L, L)] = jnp.zeros((L,), jnp.float32)
            return 0

        lax.fori_loop(0, BB, zrow, 0)
        for k0 in range(-(-STRIPE // BB)):
            nrow = min(BB, STRIPE - k0 * BB)
            pltpu.sync_copy(rows_a.at[pl.ds(0, nrow)],
                            acc_sh.at[pl.ds(s * STRIPE + k0 * BB, nrow)])
        plsc.subcore_barrier()

        stage_sx(0, sx_a)
        fire(0, idx_q, rows_a, sem_a)

        def pair(k, _):
            b = 2 * k
            stage_sx(b + 1, sx_b)
            fire(b + 1, idx_q, rows_b, sem_b)
            pltpu.make_async_copy(table_hbm.at[idx_q.at[pl.ds(0, BB)]],
                                  rows_a, sem_a).wait()
            process(b, rows_a, sx_a)

            @pl.when(k < NBLK // 2 - 1)
            def _():
                stage_sx(b + 2, sx_a)
                fire(b + 2, idx_q, rows_a, sem_a)

            pltpu.make_async_copy(table_hbm.at[idx_q.at[pl.ds(0, BB)]],
                                  rows_b, sem_b).wait()
            process(b + 1, rows_b, sx_b)
            return 0

        lax.fori_loop(0, NBLK // 2, pair, 0)
        plsc.subcore_barrier()

        @pl.when(s < NS - 1)
        def _():
            pltpu.sync_copy(
                acc_sh.at[pl.ds(s * STRIPE, STRIPE)],
                out_hbm.at[pl.ds(c * N + s * STRIPE, STRIPE),
                           pl.ds(q * QW, QW)])

        @pl.when(s == NS - 1)
        def _():
            last = N - (NS - 1) * STRIPE
            pltpu.sync_copy(
                acc_sh.at[pl.ds((NS - 1) * STRIPE, last)],
                out_hbm.at[pl.ds(c * N + (NS - 1) * STRIPE, last),
                           pl.ds(q * QW, QW)])

        plsc.subcore_barrier()


def _make_gs(bm, qstep):
    @functools.partial(
        pl.kernel,
        out_type=jax.ShapeDtypeStruct((NC * N, W), jnp.float32),
        mesh=_mesh,
        scratch_types=[
            pltpu.VMEM((BIGE,), jnp.int32),
            pltpu.VMEM((BIGE,), jnp.int32),
            pltpu.VMEM((BIGE,), jnp.int32),
            pltpu.VMEM((BIGE,), jnp.float32),
            pltpu.VMEM((BB,), jnp.int32),
            pltpu.VMEM((BB,), jnp.int32),
            pltpu.VMEM((BB, QW), jnp.float32),
            pltpu.VMEM((BB, QW), jnp.float32),
            pltpu.VMEM_SHARED((ACC_ROWS, QW), jnp.float32),
            pltpu.SemaphoreType.DMA,
            pltpu.SemaphoreType.DMA,
        ],
    )
    def k(table_hbm, eidx_hbm, en_hbm, *rest):
        _gs_body(bm, qstep, table_hbm, eidx_hbm, en_hbm, *rest)

    return k


_gs_w1 = _make_gs(2, 1)
_gs_y = _make_gs(1, N)


BLK = 400


def _h_block(p1_ref, root1_ref, bias1_ref):
    h = p1_ref[0] + p1_ref[1] + root1_ref[...] + bias1_ref[...]
    return jnp.maximum(h, 0.0).astype(jnp.bfloat16)


def _y_block(p1_ref, root1_ref, bias1_ref, w2_ref, y_ref):
    h = _h_block(p1_ref, root1_ref, bias1_ref)

    def body(r, _):
        yr = jnp.dot(h, w2_ref[r], preferred_element_type=jnp.float32)
        y_ref[r, 0] = yr[:, :QW]
        y_ref[r, 1] = yr[:, QW:]
        return 0

    lax.fori_loop(0, R, body, 0, unroll=True)


def _y_tc(p1, root1, bias1, w2_bf16):
    return pl.pallas_call(
        _y_block,
        grid=(N // BLK,),
        in_specs=[
            pl.BlockSpec((NC, BLK, H), lambda i: (0, i, 0)),
            pl.BlockSpec((BLK, H), lambda i: (i, 0)),
            pl.BlockSpec((1, H), lambda i: (0, 0)),
            pl.BlockSpec((R, H, C), lambda i: (0, 0, 0)),
        ],
        out_specs=pl.BlockSpec((R, 2, BLK, QW), lambda i: (0, 0, i, 0)),
        out_shape=jax.ShapeDtypeStruct((R, 2, N, QW), jnp.float32),
    )(p1, root1, bias1, w2_bf16)


def _tail_block(p2_ref, p1_ref, root1_ref, bias1_ref, root2_ref,
                bias2_ref, out_ref):
    h = _h_block(p1_ref, root1_ref, bias1_ref)
    o = p2_ref[0] + p2_ref[1] + jnp.dot(h, root2_ref[...],
                                        preferred_element_type=jnp.float32)
    o = o + bias2_ref[...]
    m = jnp.max(o, axis=1, keepdims=True)
    lse = jnp.log(jnp.sum(jnp.exp(o - m), axis=1, keepdims=True))
    out_ref[...] = o - m - lse


def _tail_tc(p2, p1, root1, bias1, root2_bf16, bias2):
    return pl.pallas_call(
        _tail_block,
        grid=(N // BLK,),
        in_specs=[
            pl.BlockSpec((NC, BLK, C), lambda i: (0, i, 0)),
            pl.BlockSpec((NC, BLK, H), lambda i: (0, i, 0)),
            pl.BlockSpec((BLK, H), lambda i: (i, 0)),
            pl.BlockSpec((1, H), lambda i: (0, 0)),
            pl.BlockSpec((H, C), lambda i: (0, 0)),
            pl.BlockSpec((1, C), lambda i: (0, 0)),
        ],
        out_specs=pl.BlockSpec((BLK, C), lambda i: (i, 0)),
        out_shape=jax.ShapeDtypeStruct((N, C), jnp.float32),
    )(p2, p1, root1, bias1, root2_bf16, bias2)


def kernel(edge_index, edge_attr, weight1, root1, bias1, weight2, root2, bias2):
    src = edge_index[0].astype(jnp.int32)
    dst = edge_index[1].astype(jnp.int32)
    rel = edge_attr.astype(jnp.int32)
    en = _en_sc(rel, dst)
    eidx = jnp.stack([rel, src, dst])
    eidx = jnp.pad(eidx, ((0, 0), (0, PE - E))).reshape(3 * PE)

    enp = jnp.pad(en, (0, PE - E))
    p1 = _gs_w1(weight1.reshape(R * N * 2, QW), eidx, enp).reshape(NC, N, W)
    y = _y_tc(p1, root1, bias1.reshape(1, H), weight2.astype(jnp.bfloat16))
    p2 = _gs_y(y.reshape(R * 2 * N, QW), eidx, enp).reshape(NC, N, C)
    return _tail_tc(p2, p1, root1, bias1.reshape(1, H),
                    root2.astype(jnp.bfloat16), bias2.reshape(1, C))


# speedup vs baseline: 4.2273x; 1.0197x over previous
import functools

import jax
import jax.numpy as jnp
from jax import lax
from jax.experimental import pallas as pl
from jax.experimental.pallas import tpu as pltpu
from jax.experimental.pallas import tpu_sc as plsc

N = 10000
R = 16
H = 256
C = 256
E = 160000

NC = 2
NS = 16
L = 16
EPS = E // NS
EPSC = E // (NC * NS)
W = 256
QW = W // 2
BB = 96
NBLK = 54
BIGE = NBLK * BB
PE = E + BIGE - EPSC + 64
ACC_ROWS = 10240
STRIPE = ACC_ROWS // NS

_mesh = plsc.VectorSubcoreMesh(core_axis_name="c", subcore_axis_name="s")


def _fill(ref, n, value):
    v = jnp.full((L,), value, ref.dtype)

    def body(i, _):
        ref[pl.ds(i * L, L)] = v
        return 0

    lax.fori_loop(0, n // L, body, 0)


def _en_body(rel_hbm, dst_hbm, en_hbm, rel_v, dst_v, comb_v, ones_v, cnt_v,
             hist_sh, sem):
    c = lax.axis_index("c")
    s = lax.axis_index("s")
    base = s * EPS
    _fill(cnt_v, EPS, 0.0)
    _fill(ones_v, EPS, 1.0)
    pltpu.sync_copy(cnt_v, hist_sh.at[pl.ds(s * (R * N // NS), R * N // NS)])
    pltpu.sync_copy(rel_hbm.at[pl.ds(base, EPS)], rel_v)
    pltpu.sync_copy(dst_hbm.at[pl.ds(base, EPS)], dst_v)

    def comb(i, _):
        sl = pl.ds(i * L, L)
        comb_v[sl] = rel_v[sl] * N + dst_v[sl]
        return 0

    lax.fori_loop(0, EPS // L, comb, 0)
    plsc.subcore_barrier()
    pltpu.sync_copy(ones_v, hist_sh.at[comb_v], add=True)
    plsc.subcore_barrier()

    wid = c * NS + s
    ebase = wid * EPSC
    pltpu.sync_copy(rel_hbm.at[pl.ds(ebase, EPSC)], rel_v.at[pl.ds(0, EPSC)])
    pltpu.sync_copy(dst_hbm.at[pl.ds(ebase, EPSC)], dst_v.at[pl.ds(0, EPSC)])

    def comb2(i, _):
        sl = pl.ds(i * L, L)
        comb_v[sl] = rel_v[sl] * N + dst_v[sl]
        return 0

    lax.fori_loop(0, -(-EPSC // L), comb2, 0)
    pltpu.async_copy(hist_sh.at[comb_v.at[pl.ds(0, EPSC)]],
                     cnt_v.at[pl.ds(0, EPSC)], sem).wait()

    def inv(i, _):
        sl = pl.ds(i * L, L)
        cnt_v[sl] = 1.0 / jnp.maximum(cnt_v[sl], 1.0)
        return 0

    lax.fori_loop(0, -(-EPSC // L), inv, 0)
    pltpu.sync_copy(cnt_v.at[pl.ds(0, EPSC)], en_hbm.at[pl.ds(ebase, EPSC)])


@functools.partial(
    pl.kernel,
    out_type=jax.ShapeDtypeStruct((E,), jnp.float32),
    mesh=_mesh,
    scratch_types=[
        pltpu.VMEM((EPS,), jnp.int32),
        pltpu.VMEM((EPS,), jnp.int32),
        pltpu.VMEM((EPS,), jnp.int32),
        pltpu.VMEM((EPS,), jnp.float32),
        pltpu.VMEM((EPS,), jnp.float32),
        pltpu.VMEM_SHARED((R * N,), jnp.float32),
        pltpu.SemaphoreType.DMA,
    ],
)
def _en_sc(rel_hbm, dst_hbm, *rest):
    _en_body(rel_hbm, dst_hbm, *rest)


def _gs_body(bm, qstep, table_hbm, eidx_hbm, en_hbm, out_hbm,
             idx0_v, idx1_v, dst_v, en_v, sx_a, sx_b, rows_a, rows_b,
             acc_sh, sem_a, sem_b):
    c = lax.axis_index("c")
    s = lax.axis_index("s")
    wid = c * NS + s
    ebase = wid * EPSC
    lane = lax.iota(jnp.int32, L)

    pltpu.sync_copy(eidx_hbm.at[pl.ds(ebase, BIGE)], idx0_v)
    pltpu.sync_copy(eidx_hbm.at[pl.ds(PE + ebase, BIGE)], idx1_v)
    pltpu.sync_copy(eidx_hbm.at[pl.ds(2 * PE + ebase, BIGE)], dst_v)
    pltpu.sync_copy(en_hbm.at[pl.ds(ebase, BIGE)], en_v)

    def prep(i, _):
        sl = pl.ds(i * L, L)
        g0 = idx0_v[sl] * (2 * N) + idx1_v[sl] * bm
        idx0_v[sl] = g0
        idx1_v[sl] = g0 + qstep
        pos = i * L + lane
        en_v[sl] = jnp.where(pos < EPSC, en_v[sl], 0.0)
        return 0

    lax.fori_loop(0, BIGE // L, prep, 0)

    def fire(b, idx_q, rows2, sem):
        pltpu.async_copy(table_hbm.at[idx_q.at[pl.ds(b * BB, BB)]],
                         rows2, sem)

    def stage_sx(b, sx_v):
        def cp(i, _):
            sl = pl.ds(i * L, L)
            sx_v[sl] = dst_v[pl.ds(b * BB + i * L, L)]
            return 0

        lax.fori_loop(0, BB // L, cp, 0)

    def process(b, rows2, sx_v):
        boff = b * BB

        def scale(g, _):
            ev16 = en_v[pl.ds(boff + g * L, L)]
            for r in range(L):
                ev = lax.broadcast_in_dim(ev16[r], (L,), ())
                i = g * L + r
                for j in range(QW // L):
                    sl = pl.ds(j * L, L)
                    rows2[i, sl] = rows2[i, sl] * ev
            return 0

        lax.fori_loop(0, BB // L, scale, 0)
        pltpu.sync_copy(rows2, acc_sh.at[sx_v], add=True)

    for q in range(2):
        idx_q = idx0_v if q == 0 else idx1_v

        def zrow(i, _):
            for j in range(QW // L):
                rows_a[i, pl.ds(j * L, L)] = jnp.zeros((L,), jnp.float32)
            return 0

        lax.fori_loop(0, BB, zrow, 0)
        for k0 in range(-(-STRIPE // BB)):
            nrow = min(BB, STRIPE - k0 * BB)
            pltpu.sync_copy(rows_a.at[pl.ds(0, nrow)],
                            acc_sh.at[pl.ds(s * STRIPE + k0 * BB, nrow)])
        plsc.subcore_barrier()

        stage_sx(0, sx_a)
        fire(0, idx_q, rows_a, sem_a)

        def pair(k, _):
            b = 2 * k
            stage_sx(b + 1, sx_b)
            fire(b + 1, idx_q, rows_b, sem_b)
            pltpu.make_async_copy(table_hbm.at[idx_q.at[pl.ds(0, BB)]],
                                  rows_a, sem_a).wait()
            process(b, rows_a, sx_a)

            @pl.when(k < NBLK // 2 - 1)
            def _():
                stage_sx(b + 2, sx_a)
                fire(b + 2, idx_q, rows_a, sem_a)

            pltpu.make_async_copy(table_hbm.at[idx_q.at[pl.ds(0, BB)]],
                                  rows_b, sem_b).wait()
            process(b + 1, rows_b, sx_b)
            return 0

        lax.fori_loop(0, NBLK // 2, pair, 0)
        plsc.subcore_barrier()

        @pl.when(s < NS - 1)
        def _():
            pltpu.sync_copy(
                acc_sh.at[pl.ds(s * STRIPE, STRIPE)],
                out_hbm.at[pl.ds(c * N + s * STRIPE, STRIPE),
                           pl.ds(q * QW, QW)])

        @pl.when(s == NS - 1)
        def _():
            last = N - (NS - 1) * STRIPE
            pltpu.sync_copy(
                acc_sh.at[pl.ds((NS - 1) * STRIPE, last)],
                out_hbm.at[pl.ds(c * N + (NS - 1) * STRIPE, last),
                           pl.ds(q * QW, QW)])

        plsc.subcore_barrier()


def _make_gs(bm, qstep):
    @functools.partial(
        pl.kernel,
        out_type=jax.ShapeDtypeStruct((NC * N, W), jnp.float32),
        mesh=_mesh,
        scratch_types=[
            pltpu.VMEM((BIGE,), jnp.int32),
            pltpu.VMEM((BIGE,), jnp.int32),
            pltpu.VMEM((BIGE,), jnp.int32),
            pltpu.VMEM((BIGE,), jnp.float32),
            pltpu.VMEM((BB,), jnp.int32),
            pltpu.VMEM((BB,), jnp.int32),
            pltpu.VMEM((BB, QW), jnp.float32),
            pltpu.VMEM((BB, QW), jnp.float32),
            pltpu.VMEM_SHARED((ACC_ROWS, QW), jnp.float32),
            pltpu.SemaphoreType.DMA,
            pltpu.SemaphoreType.DMA,
        ],
    )
    def k(table_hbm, eidx_hbm, en_hbm, *rest):
        _gs_body(bm, qstep, table_hbm, eidx_hbm, en_hbm, *rest)

    return k


_gs_w1 = _make_gs(2, 1)
_gs_y = _make_gs(1, N)


BLK = 400
BLKR = 2000


def _split_block(x_ref, o_ref):
    x = x_ref[...]
    o_ref[:, 0, :] = x[:, :QW]
    o_ref[:, 1, :] = x[:, QW:]


def _split_tc(table):
    return pl.pallas_call(
        _split_block,
        grid=(R * N // BLKR,),
        in_specs=[pl.BlockSpec((BLKR, W), lambda i: (i, 0))],
        out_specs=pl.BlockSpec((BLKR, 2, QW), lambda i: (i, 0, 0)),
        out_shape=jax.ShapeDtypeStruct((R * N, 2, QW), jnp.float32),
    )(table)


def _h_block(p1_ref, root1_ref, bias1_ref):
    h = p1_ref[0] + p1_ref[1] + root1_ref[...] + bias1_ref[...]
    return jnp.maximum(h, 0.0).astype(jnp.bfloat16)


def _y_block(p1_ref, root1_ref, bias1_ref, w2_ref, y_ref):
    h = _h_block(p1_ref, root1_ref, bias1_ref)

    def body(r, _):
        yr = jnp.dot(h, w2_ref[r], preferred_element_type=jnp.float32)
        y_ref[r, 0] = yr[:, :QW]
        y_ref[r, 1] = yr[:, QW:]
        return 0

    lax.fori_loop(0, R, body, 0, unroll=True)


def _y_tc(p1, root1, bias1, w2_bf16):
    return pl.pallas_call(
        _y_block,
        grid=(N // BLK,),
        in_specs=[
            pl.BlockSpec((NC, BLK, H), lambda i: (0, i, 0)),
            pl.BlockSpec((BLK, H), lambda i: (i, 0)),
            pl.BlockSpec((1, H), lambda i: (0, 0)),
            pl.BlockSpec((R, H, C), lambda i: (0, 0, 0)),
        ],
        out_specs=pl.BlockSpec((R, 2, BLK, QW), lambda i: (0, 0, i, 0)),
        out_shape=jax.ShapeDtypeStruct((R, 2, N, QW), jnp.float32),
    )(p1, root1, bias1, w2_bf16)


def _tail_block(p2_ref, p1_ref, root1_ref, bias1_ref, root2_ref,
                bias2_ref, out_ref):
    h = _h_block(p1_ref, root1_ref, bias1_ref)
    o = p2_ref[0] + p2_ref[1] + jnp.dot(h, root2_ref[...],
                                        preferred_element_type=jnp.float32)
    o = o + bias2_ref[...]
    m = jnp.max(o, axis=1, keepdims=True)
    lse = jnp.log(jnp.sum(jnp.exp(o - m), axis=1, keepdims=True))
    out_ref[...] = o - m - lse


def _tail_tc(p2, p1, root1, bias1, root2_bf16, bias2):
    return pl.pallas_call(
        _tail_block,
        grid=(N // BLK,),
        in_specs=[
            pl.BlockSpec((NC, BLK, C), lambda i: (0, i, 0)),
            pl.BlockSpec((NC, BLK, H), lambda i: (0, i, 0)),
            pl.BlockSpec((BLK, H), lambda i: (i, 0)),
            pl.BlockSpec((1, H), lambda i: (0, 0)),
            pl.BlockSpec((H, C), lambda i: (0, 0)),
            pl.BlockSpec((1, C), lambda i: (0, 0)),
        ],
        out_specs=pl.BlockSpec((BLK, C), lambda i: (i, 0)),
        out_shape=jax.ShapeDtypeStruct((N, C), jnp.float32),
    )(p2, p1, root1, bias1, root2_bf16, bias2)


def kernel(edge_index, edge_attr, weight1, root1, bias1, weight2, root2, bias2):
    src = edge_index[0].astype(jnp.int32)
    dst = edge_index[1].astype(jnp.int32)
    rel = edge_attr.astype(jnp.int32)
    en = _en_sc(rel, dst)
    eidx = jnp.stack([rel, src, dst])
    eidx = jnp.pad(eidx, ((0, 0), (0, PE - E))).reshape(3 * PE)

    enp = jnp.pad(en, (0, PE - E))
    w1t = _split_tc(weight1.reshape(R * N, W)).reshape(R * N * 2, QW)
    p1 = _gs_w1(w1t, eidx, enp).reshape(NC, N, W)
    y = _y_tc(p1, root1, bias1.reshape(1, H), weight2.astype(jnp.bfloat16))
    p2 = _gs_y(y.reshape(R * 2 * N, QW), eidx, enp).reshape(NC, N, C)
    return _tail_tc(p2, p1, root1, bias1.reshape(1, H),
                    root2.astype(jnp.bfloat16), bias2.reshape(1, C))
